# baseline (speedup 1.0000x reference)
import jax
import jax.numpy as jnp
from jax import lax
from jax.experimental import pallas as pl
from jax.experimental.pallas import tpu as pltpu
from jax.experimental.pallas import tpu_sc as plsc

_V = 1000000
_D = 32
_B = 16384
_K = 20

_NC = 2
_NS = 16
_NW = _NC * _NS
_BPW = _B // _NW
_CB = 32
_NCH = _BPW // _CB
_NNEG = _CB * _K
_NGATH = _NNEG // 128

_LOG1P_C = (
    2.4200538240037872e-09, 0.999999668889092, -0.49998875344797256,
    0.33316686590823513, -0.24865795250658715, 0.19337563668723085,
    -0.1451751324863907, 0.09470229552014076, -0.04713243998914813,
    0.015144988822244822, -0.0022880009946668264,
)


def _softplus(t):
    e = jnp.exp(-jnp.abs(t))
    p = jnp.full((16,), _LOG1P_C[-1], jnp.float32)
    for c in _LOG1P_C[-2::-1]:
        p = p * e + jnp.float32(c)
    return jnp.maximum(t, jnp.float32(0.0)) + p


def _sc_body(cflat, pflat, nflat, w_in, w_out, out,
             craw, praw, nraw, clin, plin, nlin, crows, prows, nrows,
             accv, sem):
    w = lax.axis_index("s") * _NC + lax.axis_index("c")
    iota = lax.iota(jnp.int32, 16)

    def chunk_body(i, acc):
        cb = w * _BPW + i * _CB
        nb = (w * _BPW + i * _CB) * _K
        pltpu.sync_copy(cflat.at[pl.ds(cb, _CB)], craw)
        pltpu.sync_copy(pflat.at[pl.ds(cb, _CB)], praw)
        pltpu.sync_copy(nflat.at[pl.ds(nb, _NNEG)], nraw)

        def cp_lines(t, carry):
            r = plsc.load_gather(craw, [iota + t * 16])
            plsc.store_scatter(clin, [iota + t * 16], r >> 2)
            r = plsc.load_gather(praw, [iota + t * 16])
            plsc.store_scatter(plin, [iota + t * 16], r >> 2)
            return carry

        def n_lines(t, carry):
            r = plsc.load_gather(nraw, [iota + t * 16])
            row = jnp.full((16,), t >> 3, jnp.int32)
            col = iota + ((t & 7) << 4)
            plsc.store_scatter(nlin, [row, col], r >> 2)
            return carry

        lax.fori_loop(0, _CB // 16, cp_lines, 0)
        lax.fori_loop(0, _NNEG // 16, n_lines, 0)

        cps = [pltpu.async_copy(w_in.at[clin], crows, sem),
               pltpu.async_copy(w_out.at[plin], prows, sem)]
        for j in range(_NGATH):
            cps.append(pltpu.async_copy(
                w_out.at[nlin.at[j]], nrows.at[pl.ds(j * 128, 128)], sem))
        for cp in cps:
            cp.wait()

        for g in range(_CB // 16):
            bvec = iota + g * 16
            craw16 = craw[pl.ds(g * 16, 16)]
            praw16 = praw[pl.ds(g * 16, 16)]
            ccol = (craw16 & 3) << 5
            pcol = (praw16 & 3) << 5
            nbase = bvec * _K
            ncols = []
            for k in range(_K):
                rawk = plsc.load_gather(nraw, [nbase + k])
                ncols.append((rawk & 3) << 5)

            def d_body(d, accs):
                c_d = plsc.load_gather(crows, [bvec, ccol + d])
                p_d = plsc.load_gather(prows, [bvec, pcol + d])
                new = [accs[0] + c_d * p_d]
                for k in range(_K):
                    n_d = plsc.load_gather(nrows, [nbase + k, ncols[k] + d])
                    new.append(accs[k + 1] + c_d * n_d)
                return new

            zero = jnp.zeros((16,), jnp.float32)
            accs = lax.fori_loop(0, _D, d_body, [zero] * (_K + 1))
            total = _softplus(-accs[0])
            for k in range(_K):
                total = total + _softplus(accs[k + 1])
            acc = acc + total
        return acc

    acc = lax.fori_loop(0, _NCH, chunk_body, jnp.zeros((16,), jnp.float32))
    accv[...] = acc
    pltpu.sync_copy(accv, out.at[pl.ds(w * 16, 16)])


@jax.jit
def kernel(centers, positives, negatives, W_in, W_out):
    w_in = W_in.reshape(_V // 4, 128)
    w_out = W_out.reshape(_V // 4, 128)
    nflat = negatives.reshape(_B * _K)
    mesh = plsc.VectorSubcoreMesh(core_axis_name="c", subcore_axis_name="s")
    partials = pl.kernel(
        _sc_body,
        mesh=mesh,
        compiler_params=pltpu.CompilerParams(
            needs_layout_passes=False, use_tc_tiling_on_sc=True),
        out_type=jax.ShapeDtypeStruct((_NW * 16,), jnp.float32),
        scratch_types=[
            pltpu.VMEM((_CB,), jnp.int32),
            pltpu.VMEM((_CB,), jnp.int32),
            pltpu.VMEM((_NNEG,), jnp.int32),
            pltpu.VMEM((_CB,), jnp.int32),
            pltpu.VMEM((_CB,), jnp.int32),
            pltpu.VMEM((_NGATH, 128), jnp.int32),
            pltpu.VMEM((_CB, 128), jnp.float32),
            pltpu.VMEM((_CB, 128), jnp.float32),
            pltpu.VMEM((_NNEG, 128), jnp.float32),
            pltpu.VMEM((16,), jnp.float32),
            pltpu.SemaphoreType.DMA,
        ],
    )(centers, positives, nflat, w_in, w_out)
    return jnp.sum(partials) / jnp.float32(_B)


# speedup vs baseline: 1.0876x; 1.0876x over previous
import jax
import jax.numpy as jnp
from jax import lax
from jax.experimental import pallas as pl
from jax.experimental.pallas import tpu as pltpu
from jax.experimental.pallas import tpu_sc as plsc

_V = 1000000
_D = 32
_B = 16384
_K = 20

_NC = 2
_NS = 16
_NW = _NC * _NS
_BPW = _B // _NW
_CB = 16
_NCH = _BPW // _CB
_NNEG = _CB * _K
_NPT = _BPW * _K

_LOG1P_C = (
    2.4200538240037872e-09, 0.999999668889092, -0.49998875344797256,
    0.33316686590823513, -0.24865795250658715, 0.19337563668723085,
    -0.1451751324863907, 0.09470229552014076, -0.04713243998914813,
    0.015144988822244822, -0.0022880009946668264,
)


def _softplus(t):
    e = jnp.exp(-jnp.abs(t))
    p = jnp.full((16,), _LOG1P_C[-1], jnp.float32)
    for c in _LOG1P_C[-2::-1]:
        p = p * e + jnp.float32(c)
    return jnp.maximum(t, jnp.float32(0.0)) + p


def _sc_body(cflat, pflat, nflat, w_in, w_out, out,
             craw, praw, nraw, clin, plin, nlin,
             crows0, prows0, nrows0, crows1, prows1, nrows1,
             accv, sem0, sem1):
    w = lax.axis_index("s") * _NC + lax.axis_index("c")
    iota = lax.iota(jnp.int32, 16)

    pltpu.sync_copy(cflat.at[pl.ds(w * _BPW, _BPW)], craw)
    pltpu.sync_copy(pflat.at[pl.ds(w * _BPW, _BPW)], praw)
    pltpu.sync_copy(nflat.at[pl.ds(w * _NPT, _NPT)], nraw)

    def cp_lines(t, carry):
        v = iota + t * 16
        plsc.store_scatter(clin, [v], plsc.load_gather(craw, [v]) >> 2)
        plsc.store_scatter(plin, [v], plsc.load_gather(praw, [v]) >> 2)
        return carry

    def n_lines(t, carry):
        v = iota + t * 16
        plsc.store_scatter(nlin, [v], plsc.load_gather(nraw, [v]) >> 2)
        return carry

    lax.fori_loop(0, _BPW // 16, cp_lines, 0)
    lax.fori_loop(0, _NPT // 16, n_lines, 0)

    def issue(i, bufs, sem):
        crows, prows, nrows = bufs
        nb = i * _NNEG
        return [
            pltpu.async_copy(w_in.at[clin.at[pl.ds(i * _CB, _CB)]], crows, sem),
            pltpu.async_copy(w_out.at[plin.at[pl.ds(i * _CB, _CB)]], prows, sem),
            pltpu.async_copy(w_out.at[nlin.at[pl.ds(nb, 128)]],
                             nrows.at[pl.ds(0, 128)], sem),
            pltpu.async_copy(w_out.at[nlin.at[pl.ds(nb + 128, 128)]],
                             nrows.at[pl.ds(128, 128)], sem),
            pltpu.async_copy(w_out.at[nlin.at[pl.ds(nb + 256, 64)]],
                             nrows.at[pl.ds(256, 64)], sem),
        ]

    def wait(bufs, sem):
        crows, prows, nrows = bufs
        pltpu.make_async_copy(w_in.at[pl.ds(0, _CB)], crows, sem).wait()
        pltpu.make_async_copy(w_out.at[pl.ds(0, _CB)], prows, sem).wait()
        pltpu.make_async_copy(w_out.at[pl.ds(0, _NNEG)], nrows, sem).wait()

    def compute(i, bufs, acc):
        crows, prows, nrows = bufs
        craw16 = plsc.load_gather(craw, [iota + i * _CB])
        praw16 = plsc.load_gather(praw, [iota + i * _CB])
        ccol = (craw16 & 3) << 5
        pcol = (praw16 & 3) << 5
        nbase = iota * _K
        ncols = []
        for k in range(_K):
            rawk = plsc.load_gather(nraw, [i * _NNEG + nbase + k])
            ncols.append((rawk & 3) << 5)

        def d_body(d, accs):
            c_d = plsc.load_gather(crows, [iota, ccol + d])
            p_d = plsc.load_gather(prows, [iota, pcol + d])
            new = [accs[0] + c_d * p_d]
            for k in range(_K):
                n_d = plsc.load_gather(nrows, [nbase + k, ncols[k] + d])
                new.append(accs[k + 1] + c_d * n_d)
            return new

        zero = jnp.zeros((16,), jnp.float32)
        accs = lax.fori_loop(0, _D, d_body, [zero] * (_K + 1))
        total = _softplus(-accs[0])
        for k in range(_K):
            total = total + _softplus(accs[k + 1])
        return acc + total

    bufs0 = (crows0, prows0, nrows0)
    bufs1 = (crows1, prows1, nrows1)

    issue(0, bufs0, sem0)

    def pair_body(j, acc):
        i0 = j * 2
        wait(bufs0, sem0)
        issue(i0 + 1, bufs1, sem1)
        acc = compute(i0, bufs0, acc)
        wait(bufs1, sem1)
        issue(jnp.minimum(i0 + 2, _NCH - 2), bufs0, sem0)
        acc = compute(i0 + 1, bufs1, acc)
        return acc

    acc = lax.fori_loop(0, _NCH // 2, pair_body, jnp.zeros((16,), jnp.float32))
    wait(bufs0, sem0)

    accv[...] = acc
    pltpu.sync_copy(accv, out.at[pl.ds(w * 16, 16)])


@jax.jit
def kernel(centers, positives, negatives, W_in, W_out):
    w_in = W_in.reshape(_V // 4, 128)
    w_out = W_out.reshape(_V // 4, 128)
    nflat = negatives.reshape(_B * _K)
    mesh = plsc.VectorSubcoreMesh(core_axis_name="c", subcore_axis_name="s")
    partials = pl.kernel(
        _sc_body,
        mesh=mesh,
        compiler_params=pltpu.CompilerParams(
            needs_layout_passes=False, use_tc_tiling_on_sc=True),
        out_type=jax.ShapeDtypeStruct((_NW * 16,), jnp.float32),
        scratch_types=[
            pltpu.VMEM((_BPW,), jnp.int32),
            pltpu.VMEM((_BPW,), jnp.int32),
            pltpu.VMEM((_NPT,), jnp.int32),
            pltpu.VMEM((_BPW,), jnp.int32),
            pltpu.VMEM((_BPW,), jnp.int32),
            pltpu.VMEM((_NPT,), jnp.int32),
            pltpu.VMEM((_CB, 128), jnp.float32),
            pltpu.VMEM((_CB, 128), jnp.float32),
            pltpu.VMEM((_NNEG, 128), jnp.float32),
            pltpu.VMEM((_CB, 128), jnp.float32),
            pltpu.VMEM((_CB, 128), jnp.float32),
            pltpu.VMEM((_NNEG, 128), jnp.float32),
            pltpu.VMEM((16,), jnp.float32),
            pltpu.SemaphoreType.DMA,
            pltpu.SemaphoreType.DMA,
        ],
    )(centers, positives, nflat, w_in, w_out)
    return jnp.sum(partials) / jnp.float32(_B)
